# baseline (speedup 1.0000x reference)
import functools

import jax
import jax.numpy as jnp
from jax import lax
from jax.experimental import pallas as pl
from jax.experimental.pallas import tpu as pltpu
from jax.experimental.pallas import tpu_sc as plsc

EMBED_DIM = 32
GRP = 128
NG = 32
NC = 2
NS = 16
NW = NC * NS


@jax.jit
def _sc_gather(idx2, tab2):
    S, NBT, _ = idx2.shape
    n_units = S * (NBT // NG)
    upw = n_units // NW
    nr = NBT // NG
    mesh = plsc.VectorSubcoreMesh(core_axis_name="c", subcore_axis_name="s")

    @functools.partial(
        pl.kernel,
        out_type=jax.ShapeDtypeStruct((S, 4, NBT, 8, GRP), jnp.float32),
        mesh=mesh,
        scratch_types=[
            pltpu.VMEM((NG, GRP), jnp.int32),
            pltpu.VMEM((2, GRP, EMBED_DIM), jnp.float32),
            pltpu.VMEM((2, 4, 8, GRP), jnp.float32),
            pltpu.SemaphoreType.DMA,
            pltpu.SemaphoreType.DMA,
            pltpu.SemaphoreType.DMA,
            pltpu.SemaphoreType.DMA,
        ],
        compiler_params=pltpu.CompilerParams(
            use_tc_tiling_on_sc=False, needs_layout_passes=False),
    )
    def k(idx_hbm, tab_hbm, out_hbm, idxu, rows_v, t_v, g0, g1, o0, o1):
        table = tab_hbm
        gsems = (g0, g1)
        osems = (o0, o1)
        wid = lax.axis_index("s") * NC + lax.axis_index("c")
        iota = lax.iota(jnp.int32, 16)
        rj = [iota + 16 * j for j in range(8)]
        bufv = [jnp.full((16,), b, jnp.int32) for b in range(2)]

        def fire_gather(g, buf):
            pltpu.async_copy(table.at[idxu.at[g]], rows_v.at[buf], gsems[buf])

        def wait_gather(g, buf):
            pltpu.make_async_copy(
                table.at[idxu.at[g]], rows_v.at[buf], gsems[buf]).wait()

        def transpose(buf):
            for d in range(EMBED_DIM):
                dv = jnp.full((16,), d, jnp.int32)
                dt, d8 = d // 8, d % 8
                for j in range(8):
                    v = plsc.load_gather(rows_v, [bufv[buf], rj[j], dv])
                    t_v[buf, dt, d8, 16 * j:16 * (j + 1)] = v

        def fire_writes(s, bt, buf):
            for dt in range(4):
                pltpu.async_copy(
                    t_v.at[buf, dt], out_hbm.at[s, dt, bt], osems[buf])

        def wait_writes(s, bt, buf):
            for dt in range(4):
                pltpu.make_async_copy(
                    t_v.at[buf, dt], out_hbm.at[s, dt, bt], osems[buf]).wait()

        @pl.loop(0, upw)
        def unit_loop(i):
            u = wid * upw + i
            s = u // nr
            bt0 = (u - s * nr) * NG
            pltpu.sync_copy(idx_hbm.at[s, pl.ds(bt0, NG)], idxu)
            fire_gather(0, 0)

            @pl.loop(0, NG // 2)
            def pair(j):
                gA = 2 * j
                fire_gather(gA + 1, 1)
                wait_gather(gA, 0)

                @pl.when(j > 0)
                def _():
                    wait_writes(s, bt0, 0)
                transpose(0)
                fire_writes(s, bt0 + gA, 0)

                @pl.when(j < NG // 2 - 1)
                def _():
                    fire_gather(gA + 2, 0)
                wait_gather(gA + 1, 1)

                @pl.when(j > 0)
                def _():
                    wait_writes(s, bt0, 1)
                transpose(1)
                fire_writes(s, bt0 + gA + 1, 1)

            wait_writes(s, bt0, 0)
            wait_writes(s, bt0, 1)

    return k(idx2, tab2)


def kernel(indices, table):
    Bt, S = indices.shape
    idx2 = indices.T.reshape(S, Bt // GRP, GRP).astype(jnp.int32)
    out5 = _sc_gather(idx2, table)
    return out5.transpose(2, 4, 0, 1, 3).reshape(Bt, S, EMBED_DIM)


# speedup vs baseline: 1.3796x; 1.3796x over previous
import functools

import jax
import jax.numpy as jnp
from jax import lax
from jax.experimental import pallas as pl
from jax.experimental.pallas import tpu as pltpu
from jax.experimental.pallas import tpu_sc as plsc

EMBED_DIM = 32
GRP = 128
NG = 32
NC = 2
NS = 16
NW = NC * NS


@jax.jit
def _sc_gather(idx2, tab2):
    S, NBT, _ = idx2.shape
    n_units = S * (NBT // NG)
    upw = n_units // NW
    nr = NBT // NG
    mesh = plsc.VectorSubcoreMesh(core_axis_name="c", subcore_axis_name="s")

    @functools.partial(
        pl.kernel,
        out_type=jax.ShapeDtypeStruct((S, 4, NBT, 8, GRP), jnp.float32),
        mesh=mesh,
        scratch_types=[
            pltpu.VMEM((NG, GRP), jnp.int32),
            pltpu.VMEM((2, GRP, EMBED_DIM), jnp.float32),
            pltpu.VMEM((2, EMBED_DIM, GRP), jnp.float32),
            pltpu.SemaphoreType.DMA,
            pltpu.SemaphoreType.DMA,
            pltpu.SemaphoreType.DMA,
            pltpu.SemaphoreType.DMA,
        ],
        compiler_params=pltpu.CompilerParams(
            use_tc_tiling_on_sc=False, needs_layout_passes=False),
    )
    def k(idx_hbm, tab_hbm, out_hbm, idxu, rows_v, t_v, g0, g1, o0, o1):
        table = tab_hbm
        gsems = (g0, g1)
        osems = (o0, o1)
        wid = lax.axis_index("s") * NC + lax.axis_index("c")
        iota = lax.iota(jnp.int32, 16)
        rj = [iota + 16 * j for j in range(8)]

        def fire_gather(g, buf):
            pltpu.async_copy(table.at[idxu.at[g]], rows_v.at[buf], gsems[buf])

        def wait_gather(g, buf):
            pltpu.make_async_copy(
                table.at[idxu.at[g]], rows_v.at[buf], gsems[buf]).wait()

        def transpose(buf):
            src = rows_v.at[buf]
            dst = t_v.at[buf]
            for d0 in range(EMBED_DIM):
                mv = (iota + d0) & 31
                for j in range(8):
                    v = plsc.load_gather(src, [rj[j], mv])
                    plsc.store_scatter(dst, [mv, rj[j]], v)

        def fire_writes(s, bt, buf):
            for dt in range(4):
                pltpu.async_copy(
                    t_v.at[buf, pl.ds(dt * 8, 8)], out_hbm.at[s, dt, bt],
                    osems[buf])

        def wait_writes(s, bt, buf):
            for dt in range(4):
                pltpu.make_async_copy(
                    t_v.at[buf, pl.ds(dt * 8, 8)], out_hbm.at[s, dt, bt],
                    osems[buf]).wait()

        @pl.loop(0, upw)
        def unit_loop(i):
            u = wid * upw + i
            s = u // nr
            bt0 = (u - s * nr) * NG
            pltpu.sync_copy(idx_hbm.at[s, pl.ds(bt0, NG)], idxu)
            fire_gather(0, 0)

            @pl.loop(0, NG // 2)
            def pair(j):
                gA = 2 * j
                fire_gather(gA + 1, 1)
                wait_gather(gA, 0)

                @pl.when(j > 0)
                def _():
                    wait_writes(s, bt0, 0)
                transpose(0)
                fire_writes(s, bt0 + gA, 0)

                @pl.when(j < NG // 2 - 1)
                def _():
                    fire_gather(gA + 2, 0)
                wait_gather(gA + 1, 1)

                @pl.when(j > 0)
                def _():
                    wait_writes(s, bt0, 1)
                transpose(1)
                fire_writes(s, bt0 + gA + 1, 1)

            wait_writes(s, bt0, 0)
            wait_writes(s, bt0, 1)

    return k(idx2, tab2)


def kernel(indices, table):
    Bt, S = indices.shape
    idx2 = indices.T.reshape(S, Bt // GRP, GRP).astype(jnp.int32)
    out5 = _sc_gather(idx2, table)
    return out5.transpose(2, 4, 0, 1, 3).reshape(Bt, S, EMBED_DIM)


# speedup vs baseline: 2.9795x; 2.1597x over previous
import functools

import jax
import jax.numpy as jnp
from jax import lax
from jax.experimental import pallas as pl
from jax.experimental.pallas import tpu as pltpu
from jax.experimental.pallas import tpu_sc as plsc

EMBED_DIM = 32
GRP = 128
NG = 32
NC = 2
NS = 16
NW = NC * NS


@jax.jit
def _sc_gather(idx2, tab2):
    S, NBT, _ = idx2.shape
    n_units = S * (NBT // NG)
    upw = n_units // NW
    nr = NBT // NG
    mesh = plsc.VectorSubcoreMesh(core_axis_name="c", subcore_axis_name="s")

    @functools.partial(
        pl.kernel,
        out_type=jax.ShapeDtypeStruct((S, 4, NBT, 8, GRP), jnp.float32),
        mesh=mesh,
        scratch_types=[
            pltpu.VMEM((NG, GRP), jnp.int32),
            pltpu.VMEM((2, GRP, EMBED_DIM), jnp.float32),
            pltpu.VMEM((2, EMBED_DIM, GRP), jnp.float32),
            pltpu.SemaphoreType.DMA,
            pltpu.SemaphoreType.DMA,
            pltpu.SemaphoreType.DMA,
            pltpu.SemaphoreType.DMA,
        ],
        compiler_params=pltpu.CompilerParams(
            use_tc_tiling_on_sc=False, needs_layout_passes=False),
    )
    def k(idx_hbm, tab_hbm, out_hbm, idxu, rows_v, t_v, g0, g1, o0, o1):
        table = tab_hbm
        gsems = (g0, g1)
        osems = (o0, o1)
        wid = lax.axis_index("s") * NC + lax.axis_index("c")
        iota = lax.iota(jnp.int32, 16)
        rj = [iota + 16 * j for j in range(8)]

        def fire_gather(g, buf):
            pltpu.async_copy(table.at[idxu.at[g]], rows_v.at[buf], gsems[buf])

        def wait_gather(g, buf):
            pltpu.make_async_copy(
                table.at[idxu.at[g]], rows_v.at[buf], gsems[buf]).wait()

        def transpose(buf):
            src = rows_v.at[buf]
            dst = t_v.at[buf]

            @plsc.parallel_loop(0, EMBED_DIM, carry=iota, unroll=2)
            def _(d0, mv):
                for j in range(8):
                    v = plsc.load_gather(src, [rj[j], mv])
                    plsc.store_scatter(dst, [mv, rj[j]], v)
                return (mv + 1) & 31

        def fire_writes(s, bt, buf):
            for dt in range(4):
                pltpu.async_copy(
                    t_v.at[buf, pl.ds(dt * 8, 8)], out_hbm.at[s, dt, bt],
                    osems[buf])

        def wait_writes(s, bt, buf):
            for dt in range(4):
                pltpu.make_async_copy(
                    t_v.at[buf, pl.ds(dt * 8, 8)], out_hbm.at[s, dt, bt],
                    osems[buf]).wait()

        @pl.loop(0, upw)
        def unit_loop(i):
            u = wid * upw + i
            s = u // nr
            bt0 = (u - s * nr) * NG
            pltpu.sync_copy(idx_hbm.at[s, pl.ds(bt0, NG)], idxu)
            fire_gather(0, 0)

            @pl.loop(0, NG // 2)
            def pair(j):
                gA = 2 * j
                fire_gather(gA + 1, 1)
                wait_gather(gA, 0)

                @pl.when(j > 0)
                def _():
                    wait_writes(s, bt0, 0)
                transpose(0)
                fire_writes(s, bt0 + gA, 0)

                @pl.when(j < NG // 2 - 1)
                def _():
                    fire_gather(gA + 2, 0)
                wait_gather(gA + 1, 1)

                @pl.when(j > 0)
                def _():
                    wait_writes(s, bt0, 1)
                transpose(1)
                fire_writes(s, bt0 + gA + 1, 1)

            wait_writes(s, bt0, 0)
            wait_writes(s, bt0, 1)

    return k(idx2, tab2)


def kernel(indices, table):
    Bt, S = indices.shape
    idx2 = indices.T.reshape(S, Bt // GRP, GRP).astype(jnp.int32)
    out5 = _sc_gather(idx2, table)
    return out5.transpose(2, 4, 0, 1, 3).reshape(Bt, S, EMBED_DIM)


# speedup vs baseline: 4.3302x; 1.4533x over previous
import functools

import jax
import jax.numpy as jnp
from jax import lax
from jax.experimental import pallas as pl
from jax.experimental.pallas import tpu as pltpu
from jax.experimental.pallas import tpu_sc as plsc

EMBED_DIM = 32
GRP = 128
NG = 32
NC = 2
NS = 16
NW = NC * NS


@jax.jit
def _sc_detile(tabT):
    D, V = tabT.shape
    NVT = (V + 127) // 128
    mesh = plsc.VectorSubcoreMesh(core_axis_name="c", subcore_axis_name="s")

    @functools.partial(
        pl.kernel,
        out_type=jax.ShapeDtypeStruct((NVT * 128 * D,), jnp.float32),
        mesh=mesh,
        scratch_types=[
            pltpu.VMEM((D, 128), jnp.float32),
            pltpu.VMEM((D, 128), jnp.float32),
            pltpu.VMEM((128 * D,), jnp.float32),
            pltpu.VMEM((128 * D,), jnp.float32),
            pltpu.SemaphoreType.DMA,
            pltpu.SemaphoreType.DMA,
            pltpu.SemaphoreType.DMA,
            pltpu.SemaphoreType.DMA,
        ],
        compiler_params=pltpu.CompilerParams(
            use_tc_tiling_on_sc=True, needs_layout_passes=False),
    )
    def k(tab_hbm, out_hbm, nv0, nv1, tv0, tv1, gi0, gi1, oo0, oo1):
        n_v = (nv0, nv1)
        t_v = (tv0, tv1)
        gsems = (gi0, gi1)
        osems = (oo0, oo1)
        wid = lax.axis_index("s") * NC + lax.axis_index("c")
        iota = lax.iota(jnp.int32, 16)
        vj = [iota + 16 * j for j in range(8)]
        n_pairs = (NVT + 2 * NW - 1) // (2 * NW)

        def vt_of(i):
            return wid + NW * i

        def fire_in(vt, buf):
            pltpu.async_copy(
                tab_hbm.at[:, pl.ds(vt * 128, 128)], n_v[buf], gsems[buf])

        def wait_in(vt, buf):
            pltpu.make_async_copy(
                tab_hbm.at[:, pl.ds(vt * 128, 128)], n_v[buf],
                gsems[buf]).wait()

        def transpose(buf):
            src = n_v[buf]
            dst = t_v[buf]

            @plsc.parallel_loop(0, D, carry=iota, unroll=2)
            def _(d0, mv):
                for j in range(8):
                    v = plsc.load_gather(src, [mv, vj[j]])
                    plsc.store_scatter(dst, [vj[j] * D + mv], v)
                return (mv + 1) & (D - 1)

        def fire_out(vt, buf):
            pltpu.async_copy(
                t_v[buf], out_hbm.at[pl.ds(vt * (128 * D), 128 * D)],
                osems[buf])

        def wait_out(vt, buf):
            pltpu.make_async_copy(
                t_v[buf], out_hbm.at[pl.ds(vt * (128 * D), 128 * D)],
                osems[buf]).wait()

        @pl.when(vt_of(0) < NVT)
        def _():
            fire_in(vt_of(0), 0)

        @pl.loop(0, n_pairs)
        def pair(p):
            iA = 2 * p
            vA = vt_of(iA)
            vB = vt_of(iA + 1)
            vC = vt_of(iA + 2)

            @pl.when(vB < NVT)
            def _():
                fire_in(vB, 1)

            @pl.when(vA < NVT)
            def _():
                wait_in(vA, 0)

                @pl.when(p > 0)
                def _():
                    wait_out(vA, 0)
                transpose(0)
                fire_out(vA, 0)

            @pl.when(vC < NVT)
            def _():
                fire_in(vC, 0)

            @pl.when(vB < NVT)
            def _():
                wait_in(vB, 1)

                @pl.when(p > 0)
                def _():
                    wait_out(vB, 1)
                transpose(1)
                fire_out(vB, 1)

        wait_out(0, 0)
        wait_out(0, 1)

    return k(tabT)


@jax.jit
def _sc_gather(idx2, tab2):
    S, NBT, _ = idx2.shape
    n_units = S * (NBT // NG)
    upw = n_units // NW
    nr = NBT // NG
    mesh = plsc.VectorSubcoreMesh(core_axis_name="c", subcore_axis_name="s")

    @functools.partial(
        pl.kernel,
        out_type=jax.ShapeDtypeStruct((S, 4, NBT, 8, GRP), jnp.float32),
        mesh=mesh,
        scratch_types=[
            pltpu.VMEM((NG, GRP), jnp.int32),
            pltpu.VMEM((2, GRP, EMBED_DIM), jnp.float32),
            pltpu.VMEM((2, EMBED_DIM, GRP), jnp.float32),
            pltpu.SemaphoreType.DMA,
            pltpu.SemaphoreType.DMA,
            pltpu.SemaphoreType.DMA,
            pltpu.SemaphoreType.DMA,
        ],
        compiler_params=pltpu.CompilerParams(
            use_tc_tiling_on_sc=False, needs_layout_passes=False),
    )
    def k(idx_hbm, tab_hbm, out_hbm, idxu, rows_v, t_v, g0, g1, o0, o1):
        table = tab_hbm
        gsems = (g0, g1)
        osems = (o0, o1)
        wid = lax.axis_index("s") * NC + lax.axis_index("c")
        iota = lax.iota(jnp.int32, 16)
        rj = [iota + 16 * j for j in range(8)]

        def fire_gather(g, buf):
            pltpu.async_copy(table.at[idxu.at[g]], rows_v.at[buf], gsems[buf])

        def wait_gather(g, buf):
            pltpu.make_async_copy(
                table.at[idxu.at[g]], rows_v.at[buf], gsems[buf]).wait()

        def transpose(buf):
            src = rows_v.at[buf]
            dst = t_v.at[buf]

            @plsc.parallel_loop(0, EMBED_DIM, carry=iota, unroll=2)
            def _(d0, mv):
                for j in range(8):
                    v = plsc.load_gather(src, [rj[j], mv])
                    plsc.store_scatter(dst, [mv, rj[j]], v)
                return (mv + 1) & 31

        def fire_writes(s, bt, buf):
            for dt in range(4):
                pltpu.async_copy(
                    t_v.at[buf, pl.ds(dt * 8, 8)], out_hbm.at[s, dt, bt],
                    osems[buf])

        def wait_writes(s, bt, buf):
            for dt in range(4):
                pltpu.make_async_copy(
                    t_v.at[buf, pl.ds(dt * 8, 8)], out_hbm.at[s, dt, bt],
                    osems[buf]).wait()

        @pl.loop(0, upw)
        def unit_loop(i):
            u = wid * upw + i
            s = u // nr
            bt0 = (u - s * nr) * NG
            pltpu.sync_copy(idx_hbm.at[s, pl.ds(bt0, NG)], idxu)
            fire_gather(0, 0)

            @pl.loop(0, NG // 2)
            def pair(j):
                gA = 2 * j
                fire_gather(gA + 1, 1)
                wait_gather(gA, 0)

                @pl.when(j > 0)
                def _():
                    wait_writes(s, bt0, 0)
                transpose(0)
                fire_writes(s, bt0 + gA, 0)

                @pl.when(j < NG // 2 - 1)
                def _():
                    fire_gather(gA + 2, 0)
                wait_gather(gA + 1, 1)

                @pl.when(j > 0)
                def _():
                    wait_writes(s, bt0, 1)
                transpose(1)
                fire_writes(s, bt0 + gA + 1, 1)

            wait_writes(s, bt0, 0)
            wait_writes(s, bt0, 1)

    return k(idx2, tab2)


def kernel(indices, table):
    Bt, S = indices.shape
    idx2 = indices.T.reshape(S, Bt // GRP, GRP).astype(jnp.int32)
    tab_lin = _sc_detile(table.T)
    tab2 = tab_lin.reshape(tab_lin.shape[0] // EMBED_DIM, EMBED_DIM)
    out5 = _sc_gather(idx2, tab2)
    return out5.transpose(2, 4, 0, 1, 3).reshape(Bt, S, EMBED_DIM)


# speedup vs baseline: 4.3476x; 1.0040x over previous
import functools

import jax
import jax.numpy as jnp
from jax import lax
from jax.experimental import pallas as pl
from jax.experimental.pallas import tpu as pltpu
from jax.experimental.pallas import tpu_sc as plsc

EMBED_DIM = 32
GRP = 128
NG = 32
NC = 2
NS = 16
NW = NC * NS


@jax.jit
def _sc_detile(tabT):
    D, V = tabT.shape
    NVT = (V + 127) // 128
    mesh = plsc.VectorSubcoreMesh(core_axis_name="c", subcore_axis_name="s")

    @functools.partial(
        pl.kernel,
        out_type=jax.ShapeDtypeStruct((NVT * 128 * D,), jnp.float32),
        mesh=mesh,
        scratch_types=[
            pltpu.VMEM((D, 128), jnp.float32),
            pltpu.VMEM((D, 128), jnp.float32),
            pltpu.VMEM((128 * D,), jnp.float32),
            pltpu.VMEM((128 * D,), jnp.float32),
            pltpu.SemaphoreType.DMA,
            pltpu.SemaphoreType.DMA,
            pltpu.SemaphoreType.DMA,
            pltpu.SemaphoreType.DMA,
        ],
        compiler_params=pltpu.CompilerParams(
            use_tc_tiling_on_sc=True, needs_layout_passes=False),
    )
    def k(tab_hbm, out_hbm, nv0, nv1, tv0, tv1, gi0, gi1, oo0, oo1):
        n_v = (nv0, nv1)
        t_v = (tv0, tv1)
        gsems = (gi0, gi1)
        osems = (oo0, oo1)
        wid = lax.axis_index("s") * NC + lax.axis_index("c")
        iota = lax.iota(jnp.int32, 16)
        vj = [iota + 16 * j for j in range(8)]
        n_pairs = (NVT + 2 * NW - 1) // (2 * NW)

        def vt_of(i):
            return wid + NW * i

        def fire_in(vt, buf):
            pltpu.async_copy(
                tab_hbm.at[:, pl.ds(vt * 128, 128)], n_v[buf], gsems[buf])

        def wait_in(vt, buf):
            pltpu.make_async_copy(
                tab_hbm.at[:, pl.ds(vt * 128, 128)], n_v[buf],
                gsems[buf]).wait()

        def transpose(buf):
            src = n_v[buf]
            dst = t_v[buf]

            @plsc.parallel_loop(0, D, carry=iota, unroll=2)
            def _(d0, mv):
                for j in range(8):
                    v = plsc.load_gather(src, [mv, vj[j]])
                    plsc.store_scatter(dst, [vj[j] * D + mv], v)
                return (mv + 1) & (D - 1)

        def fire_out(vt, buf):
            pltpu.async_copy(
                t_v[buf], out_hbm.at[pl.ds(vt * (128 * D), 128 * D)],
                osems[buf])

        def wait_out(vt, buf):
            pltpu.make_async_copy(
                t_v[buf], out_hbm.at[pl.ds(vt * (128 * D), 128 * D)],
                osems[buf]).wait()

        @pl.when(vt_of(0) < NVT)
        def _():
            fire_in(vt_of(0), 0)

        @pl.loop(0, n_pairs)
        def pair(p):
            iA = 2 * p
            vA = vt_of(iA)
            vB = vt_of(iA + 1)
            vC = vt_of(iA + 2)

            @pl.when(vB < NVT)
            def _():
                fire_in(vB, 1)

            @pl.when(vA < NVT)
            def _():
                wait_in(vA, 0)

                @pl.when(p > 0)
                def _():
                    wait_out(vA, 0)
                transpose(0)
                fire_out(vA, 0)

            @pl.when(vC < NVT)
            def _():
                fire_in(vC, 0)

            @pl.when(vB < NVT)
            def _():
                wait_in(vB, 1)

                @pl.when(p > 0)
                def _():
                    wait_out(vB, 1)
                transpose(1)
                fire_out(vB, 1)

        wait_out(0, 0)
        wait_out(0, 1)

    return k(tabT)


@jax.jit
def _sc_gather(idx2, tab2):
    S, NBT, _ = idx2.shape
    n_units = S * (NBT // NG)
    upw = n_units // NW
    nr = NBT // NG
    mesh = plsc.VectorSubcoreMesh(core_axis_name="c", subcore_axis_name="s")

    @functools.partial(
        pl.kernel,
        out_type=jax.ShapeDtypeStruct((S, 4, NBT, 8, GRP), jnp.float32),
        mesh=mesh,
        scratch_types=[
            pltpu.VMEM((NG, GRP), jnp.int32),
            pltpu.VMEM((2, GRP, EMBED_DIM), jnp.float32),
            pltpu.VMEM((2, EMBED_DIM, GRP), jnp.float32),
            pltpu.SemaphoreType.DMA,
            pltpu.SemaphoreType.DMA,
            pltpu.SemaphoreType.DMA,
            pltpu.SemaphoreType.DMA,
        ],
        compiler_params=pltpu.CompilerParams(
            use_tc_tiling_on_sc=False, needs_layout_passes=False),
    )
    def k(idx_hbm, tab_hbm, out_hbm, idxu, rows_v, t_v, g0, g1, o0, o1):
        table = tab_hbm
        gsems = (g0, g1)
        osems = (o0, o1)
        wid = lax.axis_index("s") * NC + lax.axis_index("c")
        iota = lax.iota(jnp.int32, 16)
        rj = [iota + 16 * j for j in range(8)]

        def fire_gather(g, buf):
            pltpu.async_copy(table.at[idxu.at[g]], rows_v.at[buf], gsems[buf])

        def wait_gather(g, buf):
            pltpu.make_async_copy(
                table.at[idxu.at[g]], rows_v.at[buf], gsems[buf]).wait()

        def transpose(buf):
            src = rows_v.at[buf]
            dst = t_v.at[buf]

            @plsc.parallel_loop(0, EMBED_DIM, carry=iota, unroll=4)
            def _(d0, mv):
                for j in range(8):
                    v = plsc.load_gather(src, [rj[j], mv])
                    plsc.store_scatter(dst, [mv, rj[j]], v)
                return (mv + 1) & 31

        def fire_writes(s, bt, buf):
            for dt in range(4):
                pltpu.async_copy(
                    t_v.at[buf, pl.ds(dt * 8, 8)], out_hbm.at[s, dt, bt],
                    osems[buf])

        def wait_writes(s, bt, buf):
            for dt in range(4):
                pltpu.make_async_copy(
                    t_v.at[buf, pl.ds(dt * 8, 8)], out_hbm.at[s, dt, bt],
                    osems[buf]).wait()

        @pl.loop(0, upw)
        def unit_loop(i):
            u = wid * upw + i
            s = u // nr
            bt0 = (u - s * nr) * NG
            pltpu.sync_copy(idx_hbm.at[s, pl.ds(bt0, NG)], idxu)
            fire_gather(0, 0)

            @pl.loop(0, NG // 2)
            def pair(j):
                gA = 2 * j
                fire_gather(gA + 1, 1)
                wait_gather(gA, 0)

                @pl.when(j > 0)
                def _():
                    wait_writes(s, bt0, 0)
                transpose(0)
                fire_writes(s, bt0 + gA, 0)

                @pl.when(j < NG // 2 - 1)
                def _():
                    fire_gather(gA + 2, 0)
                wait_gather(gA + 1, 1)

                @pl.when(j > 0)
                def _():
                    wait_writes(s, bt0, 1)
                transpose(1)
                fire_writes(s, bt0 + gA + 1, 1)

            wait_writes(s, bt0, 0)
            wait_writes(s, bt0, 1)

    return k(idx2, tab2)


def kernel(indices, table):
    Bt, S = indices.shape
    idx2 = indices.T.reshape(S, Bt // GRP, GRP).astype(jnp.int32)
    tab_lin = _sc_detile(table.T)
    tab2 = tab_lin.reshape(tab_lin.shape[0] // EMBED_DIM, EMBED_DIM)
    out5 = _sc_gather(idx2, tab2)
    return out5.transpose(2, 4, 0, 1, 3).reshape(Bt, S, EMBED_DIM)


# speedup vs baseline: 5.8611x; 1.3481x over previous
import functools

import jax
import jax.numpy as jnp
from jax import lax
from jax.experimental import pallas as pl
from jax.experimental.pallas import tpu as pltpu
from jax.experimental.pallas import tpu_sc as plsc

EMBED_DIM = 32
GRP = 128
NG = 32
NC = 2
NS = 16
NW = NC * NS


@jax.jit
def _sc_detile(tabT):
    D, V = tabT.shape
    NVT = (V + 127) // 128
    mesh = plsc.VectorSubcoreMesh(core_axis_name="c", subcore_axis_name="s")

    @functools.partial(
        pl.kernel,
        out_type=jax.ShapeDtypeStruct((NVT * 128 * D,), jnp.float32),
        mesh=mesh,
        scratch_types=[
            pltpu.VMEM((D, 128), jnp.float32),
            pltpu.VMEM((D, 128), jnp.float32),
            pltpu.VMEM((128 * D,), jnp.float32),
            pltpu.VMEM((128 * D,), jnp.float32),
            pltpu.SemaphoreType.DMA,
            pltpu.SemaphoreType.DMA,
            pltpu.SemaphoreType.DMA,
            pltpu.SemaphoreType.DMA,
        ],
        compiler_params=pltpu.CompilerParams(
            use_tc_tiling_on_sc=True, needs_layout_passes=False),
    )
    def k(tab_hbm, out_hbm, nv0, nv1, tv0, tv1, gi0, gi1, oo0, oo1):
        n_v = (nv0, nv1)
        t_v = (tv0, tv1)
        gsems = (gi0, gi1)
        osems = (oo0, oo1)
        wid = lax.axis_index("s") * NC + lax.axis_index("c")
        iota = lax.iota(jnp.int32, 16)
        vj = [iota + 16 * j for j in range(8)]
        n_pairs = (NVT + 2 * NW - 1) // (2 * NW)

        def vt_of(i):
            return wid + NW * i

        def fire_in(vt, buf):
            pltpu.async_copy(
                tab_hbm.at[:, pl.ds(vt * 128, 128)], n_v[buf], gsems[buf])

        def wait_in(vt, buf):
            pltpu.make_async_copy(
                tab_hbm.at[:, pl.ds(vt * 128, 128)], n_v[buf],
                gsems[buf]).wait()

        def transpose(buf):
            src = n_v[buf]
            dst = t_v[buf]

            @plsc.parallel_loop(0, D, carry=iota, unroll=2)
            def _(d0, mv):
                for j in range(8):
                    v = plsc.load_gather(src, [mv, vj[j]])
                    plsc.store_scatter(dst, [vj[j] * D + mv], v)
                return (mv + 1) & (D - 1)

        def fire_out(vt, buf):
            pltpu.async_copy(
                t_v[buf], out_hbm.at[pl.ds(vt * (128 * D), 128 * D)],
                osems[buf])

        def wait_out(vt, buf):
            pltpu.make_async_copy(
                t_v[buf], out_hbm.at[pl.ds(vt * (128 * D), 128 * D)],
                osems[buf]).wait()

        @pl.when(vt_of(0) < NVT)
        def _():
            fire_in(vt_of(0), 0)

        @pl.loop(0, n_pairs)
        def pair(p):
            iA = 2 * p
            vA = vt_of(iA)
            vB = vt_of(iA + 1)
            vC = vt_of(iA + 2)

            @pl.when(vB < NVT)
            def _():
                fire_in(vB, 1)

            @pl.when(vA < NVT)
            def _():
                wait_in(vA, 0)

                @pl.when(p > 0)
                def _():
                    wait_out(vA, 0)
                transpose(0)
                fire_out(vA, 0)

            @pl.when(vC < NVT)
            def _():
                fire_in(vC, 0)

            @pl.when(vB < NVT)
            def _():
                wait_in(vB, 1)

                @pl.when(p > 0)
                def _():
                    wait_out(vB, 1)
                transpose(1)
                fire_out(vB, 1)

        wait_out(0, 0)
        wait_out(0, 1)

    return k(tabT)


@jax.jit
def _sc_gather(idx2, tab2):
    S, NBT, _ = idx2.shape
    n_units = S * (NBT // NG)
    upw = n_units // NW
    nr = NBT // NG
    mesh = plsc.VectorSubcoreMesh(core_axis_name="c", subcore_axis_name="s")

    @functools.partial(
        pl.kernel,
        out_type=jax.ShapeDtypeStruct((S, 4, NBT, 8, GRP), jnp.float32),
        mesh=mesh,
        scratch_types=[
            pltpu.VMEM((NG, GRP), jnp.int32),
            pltpu.VMEM((4, GRP, EMBED_DIM), jnp.float32),
            pltpu.VMEM((4, EMBED_DIM, GRP), jnp.float32),
            pltpu.SemaphoreType.DMA,
            pltpu.SemaphoreType.DMA,
            pltpu.SemaphoreType.DMA,
            pltpu.SemaphoreType.DMA,
            pltpu.SemaphoreType.DMA,
            pltpu.SemaphoreType.DMA,
            pltpu.SemaphoreType.DMA,
            pltpu.SemaphoreType.DMA,
        ],
        compiler_params=pltpu.CompilerParams(
            use_tc_tiling_on_sc=False, needs_layout_passes=False),
    )
    def k(idx_hbm, tab_hbm, out_hbm, idxu, rows_v, t_v,
          g0, g1, g2, g3, o0, o1, o2, o3):
        table = tab_hbm
        gsems = (g0, g1, g2, g3)
        osems = (o0, o1, o2, o3)
        wid = lax.axis_index("s") * NC + lax.axis_index("c")
        iota = lax.iota(jnp.int32, 16)
        rj = [iota + 16 * j for j in range(8)]

        def fire_gather(g, buf):
            pltpu.async_copy(table.at[idxu.at[g]], rows_v.at[buf], gsems[buf])

        def wait_gather(g, buf):
            pltpu.make_async_copy(
                table.at[idxu.at[g]], rows_v.at[buf], gsems[buf]).wait()

        def transpose(buf):
            src = rows_v.at[buf]
            dst = t_v.at[buf]

            @plsc.parallel_loop(0, EMBED_DIM, carry=iota, unroll=4)
            def _(d0, mv):
                for j in range(8):
                    v = plsc.load_gather(src, [rj[j], mv])
                    plsc.store_scatter(dst, [mv, rj[j]], v)
                return (mv + 1) & 31

        def fire_writes(s, bt, buf):
            for dt in range(4):
                pltpu.async_copy(
                    t_v.at[buf, pl.ds(dt * 8, 8)], out_hbm.at[s, dt, bt],
                    osems[buf])

        def wait_writes(s, bt, buf):
            for dt in range(4):
                pltpu.make_async_copy(
                    t_v.at[buf, pl.ds(dt * 8, 8)], out_hbm.at[s, dt, bt],
                    osems[buf]).wait()

        @pl.loop(0, upw)
        def unit_loop(i):
            u = wid * upw + i
            s = u // nr
            bt0 = (u - s * nr) * NG
            pltpu.sync_copy(idx_hbm.at[s, pl.ds(bt0, NG)], idxu)
            for k in range(3):
                fire_gather(k, k)

            @pl.loop(0, NG // 4)
            def quad(q):
                for k in range(4):
                    g = 4 * q + k

                    @pl.when(g < NG - 3)
                    def _():
                        fire_gather(g + 3, (k + 3) % 4)
                    wait_gather(g, k)

                    @pl.when((i > 0) | (q > 0))
                    def _():
                        wait_writes(s, bt0, k)
                    transpose(k)
                    fire_writes(s, bt0 + g, k)

        for k in range(4):
            wait_writes(0, 0, k)

    return k(idx2, tab2)


def kernel(indices, table):
    Bt, S = indices.shape
    idx2 = indices.T.reshape(S, Bt // GRP, GRP).astype(jnp.int32)
    tab_lin = _sc_detile(table.T)
    tab2 = tab_lin.reshape(tab_lin.shape[0] // EMBED_DIM, EMBED_DIM)
    out5 = _sc_gather(idx2, tab2)
    return out5.transpose(2, 4, 0, 1, 3).reshape(Bt, S, EMBED_DIM)
